# baseline (speedup 1.0000x reference)
import jax
import jax.numpy as jnp
from jax import lax
from jax.experimental import pallas as pl
from jax.experimental.pallas import tpu as pltpu
from jax.experimental.pallas import tpu_sc as plsc

N = 1048576
NQ = 8192
NB = 2 * NQ
LAM = 0.5
GAMMA = 0.2

NC = 2
NS = 16
NW = NC * NS
CHUNK = N // NW
SUB = 8192
NSUB = CHUNK // SUB
L = 16
SLICE = NB // NS

REP = 8
ACC = REP * NB

ROWS = 1024
COLS = 1024
BROWS = 128
GRID = ROWS // BROWS


def _sc_body(scores_hbm, labels_hbm, qids_hbm, out_hbm,
             sc_v0, lb_v0, qd_v0, idx_v0, sc_v1, lb_v1, qd_v1, idx_v1,
             ones_v, zero_v, sums_sh, cnts_sh,
             in_sem0, in_sem1, st_sem0, st_sem1, z_sem):
    cid = lax.axis_index("c")
    sid = lax.axis_index("s")
    wid = cid * NS + sid
    base = wid * CHUNK

    sc_v = (sc_v0, sc_v1)
    lb_v = (lb_v0, lb_v1)
    qd_v = (qd_v0, qd_v1)
    idx_v = (idx_v0, idx_v1)
    in_sem = (in_sem0, in_sem1)
    st_sem = (st_sem0, st_sem1)

    def _issue_in(s, b):
        off = base + s * SUB
        return (
            pltpu.async_copy(scores_hbm.at[pl.ds(off, SUB)], sc_v[b], in_sem[b]),
            pltpu.async_copy(labels_hbm.at[pl.ds(off, SUB)], lb_v[b], in_sem[b]),
            pltpu.async_copy(qids_hbm.at[pl.ds(off, SUB)], qd_v[b], in_sem[b]),
        )

    pending_in = _issue_in(0, 0)

    @plsc.parallel_loop(0, SLICE // L, 1, unroll=8)
    def _zfill(i):
        zero_v[pl.ds(i * L, L)] = jnp.zeros((L,), jnp.float32)

    zdescs = []
    for r in range(REP):
        zdescs.append(pltpu.async_copy(
            zero_v, sums_sh.at[pl.ds(r * NB + sid * SLICE, SLICE)], z_sem))
        zdescs.append(pltpu.async_copy(
            zero_v, cnts_sh.at[pl.ds(r * NB + sid * SLICE, SLICE)], z_sem))

    @plsc.parallel_loop(0, SUB // L, 1, unroll=8)
    def _ofill(i):
        ones_v[pl.ds(i * L, L)] = jnp.ones((L,), jnp.float32)

    for d in zdescs:
        d.wait()
    plsc.subcore_barrier()

    roff = (lax.iota(jnp.int32, L) & (REP - 1)) * NB

    pending_st = [None, None]
    for s in range(NSUB):
        b = s & 1
        for d in pending_in:
            d.wait()
        if s + 1 < NSUB:
            if pending_st[b ^ 1] is not None:
                for d in pending_st[b ^ 1]:
                    d.wait()
                pending_st[b ^ 1] = None
            pending_in = _issue_in(s + 1, b ^ 1)

        qd_b, lb_b, idx_b = qd_v[b], lb_v[b], idx_v[b]

        @plsc.parallel_loop(0, SUB // L, 1, unroll=8)
        def _mkidx(i):
            q = qd_b[pl.ds(i * L, L)]
            lb = lb_b[pl.ds(i * L, L)]
            idx_b[pl.ds(i * L, L)] = q + (lb << 13) + roff

        pending_st[b] = (
            pltpu.async_copy(sc_v[b], sums_sh.at[idx_b], st_sem[b], add=True),
            pltpu.async_copy(ones_v, cnts_sh.at[idx_b], st_sem[b], add=True),
        )

    for b in (0, 1):
        if pending_st[b] is not None:
            for d in pending_st[b]:
                d.wait()

    plsc.subcore_barrier()

    wdescs = []
    for r in range(REP):
        wdescs.append(pltpu.async_copy(
            sums_sh.at[pl.ds(r * NB + sid * SLICE, SLICE)],
            out_hbm.at[cid, 0, r, pl.ds(sid * SLICE, SLICE)], z_sem))
        wdescs.append(pltpu.async_copy(
            cnts_sh.at[pl.ds(r * NB + sid * SLICE, SLICE)],
            out_hbm.at[cid, 1, r, pl.ds(sid * SLICE, SLICE)], z_sem))
    for d in wdescs:
        d.wait()


_sc_call = pl.kernel(
    _sc_body,
    out_type=jax.ShapeDtypeStruct((NC, 2, REP, NB), jnp.float32),
    mesh=plsc.VectorSubcoreMesh(core_axis_name="c", subcore_axis_name="s"),
    scratch_types=[
        pltpu.VMEM((SUB,), jnp.float32),
        pltpu.VMEM((SUB,), jnp.int32),
        pltpu.VMEM((SUB,), jnp.int32),
        pltpu.VMEM((SUB,), jnp.int32),
        pltpu.VMEM((SUB,), jnp.float32),
        pltpu.VMEM((SUB,), jnp.int32),
        pltpu.VMEM((SUB,), jnp.int32),
        pltpu.VMEM((SUB,), jnp.int32),
        pltpu.VMEM((SUB,), jnp.float32),
        pltpu.VMEM((SLICE,), jnp.float32),
        pltpu.VMEM_SHARED((ACC,), jnp.float32),
        pltpu.VMEM_SHARED((ACC,), jnp.float32),
        pltpu.SemaphoreType.DMA,
        pltpu.SemaphoreType.DMA,
        pltpu.SemaphoreType.DMA,
        pltpu.SemaphoreType.DMA,
        pltpu.SemaphoreType.DMA,
    ],
)


def _bce_body(scores_ref, labels_ref, out_ref):
    i = pl.program_id(0)

    @pl.when(i == 0)
    def _():
        out_ref[0, 0] = 0.0

    s = scores_ref[...]
    lb = labels_ref[...].astype(jnp.float32)
    t = lb * jnp.maximum(jnp.log(s), -100.0) \
        + (1.0 - lb) * jnp.maximum(jnp.log1p(-s), -100.0)
    out_ref[0, 0] += jnp.sum(t)


def _bce_call(scores2, labels2):
    return pl.pallas_call(
        _bce_body,
        grid=(GRID,),
        in_specs=[
            pl.BlockSpec((BROWS, COLS), lambda i: (i, 0)),
            pl.BlockSpec((BROWS, COLS), lambda i: (i, 0)),
        ],
        out_specs=pl.BlockSpec(memory_space=pltpu.SMEM),
        out_shape=jax.ShapeDtypeStruct((1, 1), jnp.float32),
    )(scores2, labels2)


def _comb_body(sums_ref, cnts_ref, bce_ref, out_ref):
    sums = jnp.sum(sums_ref[...], axis=0, keepdims=True)
    cnts = jnp.sum(cnts_ref[...], axis=0, keepdims=True)
    neg_s = sums[:, :NQ]
    pos_s = sums[:, NQ:]
    neg_c = cnts[:, :NQ]
    pos_c = cnts[:, NQ:]
    valid = (pos_c > 0.0) & (neg_c > 0.0)
    pos_m = pos_s / jnp.maximum(pos_c, 1.0)
    neg_m = neg_s / jnp.maximum(neg_c, 1.0)
    delta = pos_m - neg_m
    terms = jnp.where(valid, jnp.maximum(GAMMA - delta, 0.0), 0.0)
    ng = jnp.sum(valid.astype(jnp.float32))
    ldc = jnp.where(ng > 0.0, jnp.sum(terms) / jnp.maximum(ng, 1.0), 0.0)
    lce = -bce_ref[0, 0] / N
    out_ref[0, 0] = lce + LAM * ldc


def _comb_call(sums_p, cnts_p, bce):
    return pl.pallas_call(
        _comb_body,
        in_specs=[
            pl.BlockSpec((NC * REP, NB), lambda: (0, 0)),
            pl.BlockSpec((NC * REP, NB), lambda: (0, 0)),
            pl.BlockSpec(memory_space=pltpu.SMEM),
        ],
        out_specs=pl.BlockSpec(memory_space=pltpu.SMEM),
        out_shape=jax.ShapeDtypeStruct((1, 1), jnp.float32),
    )(sums_p, cnts_p, bce)


def kernel(scores, labels, qids):
    labels_i = labels.astype(jnp.int32)
    qids_i = qids.astype(jnp.int32)
    acc = _sc_call(scores, labels_i, qids_i)
    sums_p = acc[:, 0].reshape(NC * REP, NB)
    cnts_p = acc[:, 1].reshape(NC * REP, NB)
    scores2 = scores.reshape(ROWS, COLS)
    labels2 = labels_i.reshape(ROWS, COLS)
    bce = _bce_call(scores2, labels2)
    out = _comb_call(sums_p, cnts_p, bce)
    return out[0, 0]


# speedup vs baseline: 1.4247x; 1.4247x over previous
import jax
import jax.numpy as jnp
from jax import lax
from jax.experimental import pallas as pl
from jax.experimental.pallas import tpu as pltpu
from jax.experimental.pallas import tpu_sc as plsc

N = 1048576
NQ = 8192
NB = 2 * NQ
LAM = 0.5
GAMMA = 0.2

NC = 2
NS = 16
NW = NC * NS
CHUNK = N // NW
SUB = 4096
NSUB = CHUNK // SUB
L = 16

RL = 2
LACC = RL * NB

ROWS = 1024
COLS = 1024
BROWS = 128
GRID = ROWS // BROWS


def _sc_body(scores_hbm, labels_hbm, qids_hbm, out_hbm,
             sc_v0, lb_v0, qd_v0, sc_v1, lb_v1, qd_v1,
             lsum_v, lcnt_v,
             in_sem0, in_sem1, out_sem):
    cid = lax.axis_index("c")
    sid = lax.axis_index("s")
    wid = cid * NS + sid
    base = wid * CHUNK

    sc_v = (sc_v0, sc_v1)
    lb_v = (lb_v0, lb_v1)
    qd_v = (qd_v0, qd_v1)
    in_sem = (in_sem0, in_sem1)

    def _issue_in(s, b):
        off = base + s * SUB
        return (
            pltpu.async_copy(scores_hbm.at[pl.ds(off, SUB)], sc_v[b], in_sem[b]),
            pltpu.async_copy(labels_hbm.at[pl.ds(off, SUB)], lb_v[b], in_sem[b]),
            pltpu.async_copy(qids_hbm.at[pl.ds(off, SUB)], qd_v[b], in_sem[b]),
        )

    pending_in = _issue_in(0, 0)

    @plsc.parallel_loop(0, LACC // L, 1, unroll=8)
    def _zfill(i):
        lsum_v[pl.ds(i * L, L)] = jnp.zeros((L,), jnp.float32)
        lcnt_v[pl.ds(i * L, L)] = jnp.zeros((L,), jnp.float32)

    lane = lax.iota(jnp.int32, L)
    roff = (lane & (RL - 1)) * NB
    ones_f = jnp.ones((L,), jnp.float32)

    for s in range(NSUB):
        b = s & 1
        for d in pending_in:
            d.wait()
        if s + 1 < NSUB:
            pending_in = _issue_in(s + 1, b ^ 1)

        qd_b, lb_b, sc_b = qd_v[b], lb_v[b], sc_v[b]

        @plsc.parallel_loop(0, SUB // L, 1, unroll=8)
        def _accum(i):
            q = qd_b[pl.ds(i * L, L)]
            lb = lb_b[pl.ds(i * L, L)]
            sv = sc_b[pl.ds(i * L, L)]
            bidx = q + (lb << 13) + roff
            plsc.addupdate_scatter(lsum_v, [bidx], sv)
            plsc.addupdate_scatter(lcnt_v, [bidx], ones_f)

    @plsc.parallel_loop(0, NB // L, 1, unroll=8)
    def _fold(i):
        lsum_v[pl.ds(i * L, L)] += lsum_v[pl.ds(NB + i * L, L)]
        lcnt_v[pl.ds(i * L, L)] += lcnt_v[pl.ds(NB + i * L, L)]

    d1 = pltpu.async_copy(lsum_v.at[pl.ds(0, NB)],
                          out_hbm.at[cid, 0, sid, :], out_sem)
    d2 = pltpu.async_copy(lcnt_v.at[pl.ds(0, NB)],
                          out_hbm.at[cid, 1, sid, :], out_sem)
    d1.wait()
    d2.wait()


_sc_call = pl.kernel(
    _sc_body,
    out_type=jax.ShapeDtypeStruct((NC, 2, NS, NB), jnp.float32),
    mesh=plsc.VectorSubcoreMesh(core_axis_name="c", subcore_axis_name="s"),
    compiler_params=pltpu.CompilerParams(needs_layout_passes=False),
    scratch_types=[
        pltpu.VMEM((SUB,), jnp.float32),
        pltpu.VMEM((SUB,), jnp.int32),
        pltpu.VMEM((SUB,), jnp.int32),
        pltpu.VMEM((SUB,), jnp.float32),
        pltpu.VMEM((SUB,), jnp.int32),
        pltpu.VMEM((SUB,), jnp.int32),
        pltpu.VMEM((LACC,), jnp.float32),
        pltpu.VMEM((LACC,), jnp.float32),
        pltpu.SemaphoreType.DMA,
        pltpu.SemaphoreType.DMA,
        pltpu.SemaphoreType.DMA,
    ],
)


def _bce_body(scores_ref, labels_ref, out_ref):
    i = pl.program_id(0)

    @pl.when(i == 0)
    def _():
        out_ref[0, 0] = 0.0

    s = scores_ref[...]
    lb = labels_ref[...].astype(jnp.float32)
    t = lb * jnp.maximum(jnp.log(s), -100.0) \
        + (1.0 - lb) * jnp.maximum(jnp.log1p(-s), -100.0)
    out_ref[0, 0] += jnp.sum(t)


def _bce_call(scores2, labels2):
    return pl.pallas_call(
        _bce_body,
        grid=(GRID,),
        in_specs=[
            pl.BlockSpec((BROWS, COLS), lambda i: (i, 0)),
            pl.BlockSpec((BROWS, COLS), lambda i: (i, 0)),
        ],
        out_specs=pl.BlockSpec(memory_space=pltpu.SMEM),
        out_shape=jax.ShapeDtypeStruct((1, 1), jnp.float32),
    )(scores2, labels2)


def _comb_body(sums_ref, cnts_ref, bce_ref, out_ref):
    sums = jnp.sum(sums_ref[...], axis=0, keepdims=True)
    cnts = jnp.sum(cnts_ref[...], axis=0, keepdims=True)
    neg_s = sums[:, :NQ]
    pos_s = sums[:, NQ:]
    neg_c = cnts[:, :NQ]
    pos_c = cnts[:, NQ:]
    valid = (pos_c > 0.0) & (neg_c > 0.0)
    pos_m = pos_s / jnp.maximum(pos_c, 1.0)
    neg_m = neg_s / jnp.maximum(neg_c, 1.0)
    delta = pos_m - neg_m
    terms = jnp.where(valid, jnp.maximum(GAMMA - delta, 0.0), 0.0)
    ng = jnp.sum(valid.astype(jnp.float32))
    ldc = jnp.where(ng > 0.0, jnp.sum(terms) / jnp.maximum(ng, 1.0), 0.0)
    lce = -bce_ref[0, 0] / N
    out_ref[0, 0] = lce + LAM * ldc


def _comb_call(sums_p, cnts_p, bce):
    return pl.pallas_call(
        _comb_body,
        in_specs=[
            pl.BlockSpec((NC * NS, NB), lambda: (0, 0)),
            pl.BlockSpec((NC * NS, NB), lambda: (0, 0)),
            pl.BlockSpec(memory_space=pltpu.SMEM),
        ],
        out_specs=pl.BlockSpec(memory_space=pltpu.SMEM),
        out_shape=jax.ShapeDtypeStruct((1, 1), jnp.float32),
    )(sums_p, cnts_p, bce)


def kernel(scores, labels, qids):
    labels_i = labels.astype(jnp.int32)
    qids_i = qids.astype(jnp.int32)
    acc = _sc_call(scores, labels_i, qids_i)
    sums_p = acc[:, 0].reshape(NC * NS, NB)
    cnts_p = acc[:, 1].reshape(NC * NS, NB)
    scores2 = scores.reshape(ROWS, COLS)
    labels2 = labels_i.reshape(ROWS, COLS)
    bce = _bce_call(scores2, labels2)
    out = _comb_call(sums_p, cnts_p, bce)
    return out[0, 0]


# speedup vs baseline: 1.5807x; 1.1095x over previous
import jax
import jax.numpy as jnp
from jax import lax
from jax.experimental import pallas as pl
from jax.experimental.pallas import tpu as pltpu
from jax.experimental.pallas import tpu_sc as plsc

N = 1048576
NQ = 8192
NB = 2 * NQ
LAM = 0.5
GAMMA = 0.2

NC = 2
NS = 16
NW = NC * NS
CHUNK = N // NW
SUB = 8192
NSUB = CHUNK // SUB
L = 16

ROWS = 1024
COLS = 1024
BROWS = 128
GRID = ROWS // BROWS


def _sc_body(scores_hbm, labels_hbm, qids_hbm, sums_hbm, cnts_hbm,
             sc_v0, lb_v0, qd_v0, sc_v1, lb_v1, qd_v1,
             lsum_v, lcnt_v,
             in_sem0, in_sem1, out_sem):
    cid = lax.axis_index("c")
    sid = lax.axis_index("s")
    wid = cid * NS + sid
    base = wid * CHUNK

    sc_v = (sc_v0, sc_v1)
    lb_v = (lb_v0, lb_v1)
    qd_v = (qd_v0, qd_v1)
    in_sem = (in_sem0, in_sem1)

    def _issue_in(s, b):
        off = base + s * SUB
        return (
            pltpu.async_copy(scores_hbm.at[pl.ds(off, SUB)], sc_v[b], in_sem[b]),
            pltpu.async_copy(labels_hbm.at[pl.ds(off, SUB)], lb_v[b], in_sem[b]),
            pltpu.async_copy(qids_hbm.at[pl.ds(off, SUB)], qd_v[b], in_sem[b]),
        )

    pending_in = _issue_in(0, 0)

    @plsc.parallel_loop(0, NB // L, 1, unroll=8)
    def _zfill(i):
        lsum_v[pl.ds(i * L, L)] = jnp.zeros((L,), jnp.float32)
        lcnt_v[pl.ds(i * L, L)] = jnp.zeros((L,), jnp.float32)

    lane = lax.iota(jnp.int32, L)
    ones_f = jnp.ones((L,), jnp.float32)
    in_b = "wrap"
    perm1 = lane ^ 1
    perm2 = lane ^ 2
    perm4 = lane ^ 4

    for s in range(NSUB):
        b = s & 1
        for d in pending_in:
            d.wait()
        if s + 1 < NSUB:
            pending_in = _issue_in(s + 1, b ^ 1)

        qd_b, lb_b, sc_b = qd_v[b], lb_v[b], sc_v[b]

        @plsc.parallel_loop(0, SUB // L, 1, unroll=8)
        def _accum(i):
            q = qd_b[pl.ds(i * L, L)]
            lb = lb_b[pl.ds(i * L, L)]
            sv = sc_b[pl.ds(i * L, L)]
            k = q + (lb << 13)
            cv = ones_f + 0.0 * sv
            keep = jnp.where(k == k, 1, 0)
            for perm, bit in ((perm1, 1), (perm2, 2), (perm4, 4)):
                kp = jnp.take(k, perm, mode=in_b)
                sp = jnp.take(sv, perm, mode=in_b)
                cp = jnp.take(cv, perm, mode=in_b)
                mp = jnp.take(keep, perm, mode=in_b)
                eq = (k == kp) & (keep == 1) & (mp == 1)
                sv = sv + jnp.where(eq, sp, 0.0)
                cv = cv + jnp.where(eq, cp, 0.0)
                keep = jnp.where(eq & ((lane & bit) != 0), 0, keep)
            live = keep == 1
            plsc.addupdate_scatter(lsum_v, [k], sv, mask=live)
            plsc.addupdate_scatter(lcnt_v, [k], cv, mask=live)

    d1 = pltpu.async_copy(lsum_v, sums_hbm.at[cid, sid, :], out_sem)
    d2 = pltpu.async_copy(lcnt_v, cnts_hbm.at[cid, sid, :], out_sem)
    d1.wait()
    d2.wait()


_sc_call = pl.kernel(
    _sc_body,
    out_type=(jax.ShapeDtypeStruct((NC, NS, NB), jnp.float32),
              jax.ShapeDtypeStruct((NC, NS, NB), jnp.float32)),
    mesh=plsc.VectorSubcoreMesh(core_axis_name="c", subcore_axis_name="s"),
    compiler_params=pltpu.CompilerParams(needs_layout_passes=False),
    scratch_types=[
        pltpu.VMEM((SUB,), jnp.float32),
        pltpu.VMEM((SUB,), jnp.int32),
        pltpu.VMEM((SUB,), jnp.int32),
        pltpu.VMEM((SUB,), jnp.float32),
        pltpu.VMEM((SUB,), jnp.int32),
        pltpu.VMEM((SUB,), jnp.int32),
        pltpu.VMEM((NB,), jnp.float32),
        pltpu.VMEM((NB,), jnp.float32),
        pltpu.SemaphoreType.DMA,
        pltpu.SemaphoreType.DMA,
        pltpu.SemaphoreType.DMA,
    ],
)


def _bce_body(scores_ref, labels_ref, out_ref):
    i = pl.program_id(0)

    @pl.when(i == 0)
    def _():
        out_ref[0, 0] = 0.0

    s = scores_ref[...]
    lb = labels_ref[...].astype(jnp.float32)
    t = lb * jnp.maximum(jnp.log(s), -100.0) \
        + (1.0 - lb) * jnp.maximum(jnp.log1p(-s), -100.0)
    out_ref[0, 0] += jnp.sum(t)


def _bce_call(scores2, labels2):
    return pl.pallas_call(
        _bce_body,
        grid=(GRID,),
        in_specs=[
            pl.BlockSpec((BROWS, COLS), lambda i: (i, 0)),
            pl.BlockSpec((BROWS, COLS), lambda i: (i, 0)),
        ],
        out_specs=pl.BlockSpec(memory_space=pltpu.SMEM),
        out_shape=jax.ShapeDtypeStruct((1, 1), jnp.float32),
    )(scores2, labels2)


def _comb_body(sums_ref, cnts_ref, bce_ref, out_ref):
    sums = jnp.sum(sums_ref[...], axis=0, keepdims=True)
    cnts = jnp.sum(cnts_ref[...], axis=0, keepdims=True)
    neg_s = sums[:, :NQ]
    pos_s = sums[:, NQ:]
    neg_c = cnts[:, :NQ]
    pos_c = cnts[:, NQ:]
    valid = (pos_c > 0.0) & (neg_c > 0.0)
    pos_m = pos_s / jnp.maximum(pos_c, 1.0)
    neg_m = neg_s / jnp.maximum(neg_c, 1.0)
    delta = pos_m - neg_m
    terms = jnp.where(valid, jnp.maximum(GAMMA - delta, 0.0), 0.0)
    ng = jnp.sum(valid.astype(jnp.float32))
    ldc = jnp.where(ng > 0.0, jnp.sum(terms) / jnp.maximum(ng, 1.0), 0.0)
    lce = -bce_ref[0, 0] / N
    out_ref[0, 0] = lce + LAM * ldc


def _comb_call(sums_p, cnts_p, bce):
    return pl.pallas_call(
        _comb_body,
        in_specs=[
            pl.BlockSpec((NC * NS, NB), lambda: (0, 0)),
            pl.BlockSpec((NC * NS, NB), lambda: (0, 0)),
            pl.BlockSpec(memory_space=pltpu.SMEM),
        ],
        out_specs=pl.BlockSpec(memory_space=pltpu.SMEM),
        out_shape=jax.ShapeDtypeStruct((1, 1), jnp.float32),
    )(sums_p, cnts_p, bce)


def kernel(scores, labels, qids):
    labels_i = labels.astype(jnp.int32)
    qids_i = qids.astype(jnp.int32)
    sums_o, cnts_o = _sc_call(scores, labels_i, qids_i)
    sums_p = sums_o.reshape(NC * NS, NB)
    cnts_p = cnts_o.reshape(NC * NS, NB)
    scores2 = scores.reshape(ROWS, COLS)
    labels2 = labels_i.reshape(ROWS, COLS)
    bce = _bce_call(scores2, labels2)
    out = _comb_call(sums_p, cnts_p, bce)
    return out[0, 0]


# speedup vs baseline: 1.6735x; 1.0587x over previous
import jax
import jax.numpy as jnp
from jax import lax
from jax.experimental import pallas as pl
from jax.experimental.pallas import tpu as pltpu
from jax.experimental.pallas import tpu_sc as plsc

N = 1048576
NQ = 8192
NB = 2 * NQ
LAM = 0.5
GAMMA = 0.2

NC = 2
NS = 16
NW = NC * NS
CHUNK = N // NW
SUB = 8192
NSUB = CHUNK // SUB
L = 16

ROWS = 1024
COLS = 1024
BROWS = 128
GRID = ROWS // BROWS


def _sc_body(scores_hbm, labels_hbm, qids_hbm, sums_hbm, cnts_hbm,
             sc_v0, lb_v0, qd_v0, sc_v1, lb_v1, qd_v1,
             lsum_v, lcnt_v,
             in_sem0, in_sem1, out_sem):
    cid = lax.axis_index("c")
    sid = lax.axis_index("s")
    wid = cid * NS + sid
    base = wid * CHUNK

    sc_v = (sc_v0, sc_v1)
    lb_v = (lb_v0, lb_v1)
    qd_v = (qd_v0, qd_v1)
    in_sem = (in_sem0, in_sem1)

    def _issue_in(s, b):
        off = base + s * SUB
        return (
            pltpu.async_copy(scores_hbm.at[pl.ds(off, SUB)], sc_v[b], in_sem[b]),
            pltpu.async_copy(labels_hbm.at[pl.ds(off, SUB)], lb_v[b], in_sem[b]),
            pltpu.async_copy(qids_hbm.at[pl.ds(off, SUB)], qd_v[b], in_sem[b]),
        )

    pending_in = _issue_in(0, 0)

    @plsc.parallel_loop(0, NB // L, 1, unroll=8)
    def _zfill(i):
        lsum_v[pl.ds(i * L, L)] = jnp.zeros((L,), jnp.float32)
        lcnt_v[pl.ds(i * L, L)] = jnp.zeros((L,), jnp.float32)

    lane = lax.iota(jnp.int32, L)
    ones_f = jnp.ones((L,), jnp.float32)
    in_b = "wrap"
    perm1 = lane ^ 1
    perm2 = lane ^ 2
    perm4 = lane ^ 4

    for s in range(NSUB):
        b = s & 1
        for d in pending_in:
            d.wait()
        if s + 1 < NSUB:
            pending_in = _issue_in(s + 1, b ^ 1)

        qd_b, lb_b, sc_b = qd_v[b], lb_v[b], sc_v[b]

        @plsc.parallel_loop(0, SUB // L, 1, unroll=8)
        def _accum(i):
            q = qd_b[pl.ds(i * L, L)]
            lb = lb_b[pl.ds(i * L, L)]
            sv = sc_b[pl.ds(i * L, L)]
            k = q + (lb << 13)
            cv = ones_f + 0.0 * sv
            keep = jnp.where(k == k, 1, 0)
            for perm, bit in ((perm1, 1), (perm2, 2)):
                kp = jnp.take(k, perm, mode=in_b)
                sp = jnp.take(sv, perm, mode=in_b)
                cp = jnp.take(cv, perm, mode=in_b)
                mp = jnp.take(keep, perm, mode=in_b)
                eq = (k == kp) & (keep == 1) & (mp == 1)
                sv = sv + jnp.where(eq, sp, 0.0)
                cv = cv + jnp.where(eq, cp, 0.0)
                keep = jnp.where(eq & ((lane & bit) != 0), 0, keep)
            live = keep == 1
            plsc.addupdate_scatter(lsum_v, [k], sv, mask=live)
            plsc.addupdate_scatter(lcnt_v, [k], cv, mask=live)

    d1 = pltpu.async_copy(lsum_v, sums_hbm.at[cid, sid, :], out_sem)
    d2 = pltpu.async_copy(lcnt_v, cnts_hbm.at[cid, sid, :], out_sem)
    d1.wait()
    d2.wait()


_sc_call = pl.kernel(
    _sc_body,
    out_type=(jax.ShapeDtypeStruct((NC, NS, NB), jnp.float32),
              jax.ShapeDtypeStruct((NC, NS, NB), jnp.float32)),
    mesh=plsc.VectorSubcoreMesh(core_axis_name="c", subcore_axis_name="s"),
    compiler_params=pltpu.CompilerParams(needs_layout_passes=False),
    scratch_types=[
        pltpu.VMEM((SUB,), jnp.float32),
        pltpu.VMEM((SUB,), jnp.int32),
        pltpu.VMEM((SUB,), jnp.int32),
        pltpu.VMEM((SUB,), jnp.float32),
        pltpu.VMEM((SUB,), jnp.int32),
        pltpu.VMEM((SUB,), jnp.int32),
        pltpu.VMEM((NB,), jnp.float32),
        pltpu.VMEM((NB,), jnp.float32),
        pltpu.SemaphoreType.DMA,
        pltpu.SemaphoreType.DMA,
        pltpu.SemaphoreType.DMA,
    ],
)


def _bce_body(scores_ref, labels_ref, out_ref):
    i = pl.program_id(0)

    @pl.when(i == 0)
    def _():
        out_ref[0, 0] = 0.0

    s = scores_ref[...]
    lb = labels_ref[...].astype(jnp.float32)
    t = lb * jnp.maximum(jnp.log(s), -100.0) \
        + (1.0 - lb) * jnp.maximum(jnp.log1p(-s), -100.0)
    out_ref[0, 0] += jnp.sum(t)


def _bce_call(scores2, labels2):
    return pl.pallas_call(
        _bce_body,
        grid=(GRID,),
        in_specs=[
            pl.BlockSpec((BROWS, COLS), lambda i: (i, 0)),
            pl.BlockSpec((BROWS, COLS), lambda i: (i, 0)),
        ],
        out_specs=pl.BlockSpec(memory_space=pltpu.SMEM),
        out_shape=jax.ShapeDtypeStruct((1, 1), jnp.float32),
    )(scores2, labels2)


def _comb_body(sums_ref, cnts_ref, bce_ref, out_ref):
    sums = jnp.sum(sums_ref[...], axis=0, keepdims=True)
    cnts = jnp.sum(cnts_ref[...], axis=0, keepdims=True)
    neg_s = sums[:, :NQ]
    pos_s = sums[:, NQ:]
    neg_c = cnts[:, :NQ]
    pos_c = cnts[:, NQ:]
    valid = (pos_c > 0.0) & (neg_c > 0.0)
    pos_m = pos_s / jnp.maximum(pos_c, 1.0)
    neg_m = neg_s / jnp.maximum(neg_c, 1.0)
    delta = pos_m - neg_m
    terms = jnp.where(valid, jnp.maximum(GAMMA - delta, 0.0), 0.0)
    ng = jnp.sum(valid.astype(jnp.float32))
    ldc = jnp.where(ng > 0.0, jnp.sum(terms) / jnp.maximum(ng, 1.0), 0.0)
    lce = -bce_ref[0, 0] / N
    out_ref[0, 0] = lce + LAM * ldc


def _comb_call(sums_p, cnts_p, bce):
    return pl.pallas_call(
        _comb_body,
        in_specs=[
            pl.BlockSpec((NC * NS, NB), lambda: (0, 0)),
            pl.BlockSpec((NC * NS, NB), lambda: (0, 0)),
            pl.BlockSpec(memory_space=pltpu.SMEM),
        ],
        out_specs=pl.BlockSpec(memory_space=pltpu.SMEM),
        out_shape=jax.ShapeDtypeStruct((1, 1), jnp.float32),
    )(sums_p, cnts_p, bce)


def kernel(scores, labels, qids):
    labels_i = labels.astype(jnp.int32)
    qids_i = qids.astype(jnp.int32)
    sums_o, cnts_o = _sc_call(scores, labels_i, qids_i)
    sums_p = sums_o.reshape(NC * NS, NB)
    cnts_p = cnts_o.reshape(NC * NS, NB)
    scores2 = scores.reshape(ROWS, COLS)
    labels2 = labels_i.reshape(ROWS, COLS)
    bce = _bce_call(scores2, labels2)
    out = _comb_call(sums_p, cnts_p, bce)
    return out[0, 0]


# speedup vs baseline: 1.7084x; 1.0208x over previous
import jax
import jax.numpy as jnp
from jax import lax
from jax.experimental import pallas as pl
from jax.experimental.pallas import tpu as pltpu
from jax.experimental.pallas import tpu_sc as plsc

N = 1048576
NQ = 8192
NB = 2 * NQ
LAM = 0.5
GAMMA = 0.2

NC = 2
NS = 16
NW = NC * NS
CHUNK = N // NW
SUB = 8192
NSUB = CHUNK // SUB
L = 16

ROWS = 1024
COLS = 1024
BROWS = 128
GRID = ROWS // BROWS


def _sc_body(scores_hbm, labels_hbm, qids_hbm, sums_hbm, cnts_hbm,
             sc_v0, lb_v0, qd_v0, sc_v1, lb_v1, qd_v1,
             lsum_v, lcnt_v,
             in_sem0, in_sem1, out_sem):
    cid = lax.axis_index("c")
    sid = lax.axis_index("s")
    wid = cid * NS + sid
    base = wid * CHUNK

    sc_v = (sc_v0, sc_v1)
    lb_v = (lb_v0, lb_v1)
    qd_v = (qd_v0, qd_v1)
    in_sem = (in_sem0, in_sem1)

    def _issue_in(s, b):
        off = base + s * SUB
        return (
            pltpu.async_copy(scores_hbm.at[pl.ds(off, SUB)], sc_v[b], in_sem[b]),
            pltpu.async_copy(labels_hbm.at[pl.ds(off, SUB)], lb_v[b], in_sem[b]),
            pltpu.async_copy(qids_hbm.at[pl.ds(off, SUB)], qd_v[b], in_sem[b]),
        )

    pending_in = _issue_in(0, 0)

    @plsc.parallel_loop(0, NB // L, 1, unroll=8)
    def _zfill(i):
        lsum_v[pl.ds(i * L, L)] = jnp.zeros((L,), jnp.float32)
        lcnt_v[pl.ds(i * L, L)] = jnp.zeros((L,), jnp.float32)

    lane = lax.iota(jnp.int32, L)
    ones_f = jnp.ones((L,), jnp.float32)
    in_b = "wrap"
    perm1 = lane ^ 1
    perm2 = lane ^ 2
    perm4 = lane ^ 4

    for s in range(NSUB):
        b = s & 1
        for d in pending_in:
            d.wait()
        if s + 1 < NSUB:
            pending_in = _issue_in(s + 1, b ^ 1)

        qd_b, lb_b, sc_b = qd_v[b], lb_v[b], sc_v[b]

        @plsc.parallel_loop(0, SUB // L, 1, unroll=8)
        def _accum(i):
            q = qd_b[pl.ds(i * L, L)]
            lb = lb_b[pl.ds(i * L, L)]
            sv = sc_b[pl.ds(i * L, L)]
            k = q + (lb << 13)
            cv = ones_f + 0.0 * sv
            keep = jnp.where(k == k, 1, 0)
            for perm, bit in ((perm1, 1),):
                kp = jnp.take(k, perm, mode=in_b)
                sp = jnp.take(sv, perm, mode=in_b)
                cp = jnp.take(cv, perm, mode=in_b)
                mp = jnp.take(keep, perm, mode=in_b)
                eq = (k == kp) & (keep == 1) & (mp == 1)
                sv = sv + jnp.where(eq, sp, 0.0)
                cv = cv + jnp.where(eq, cp, 0.0)
                keep = jnp.where(eq & ((lane & bit) != 0), 0, keep)
            live = keep == 1
            plsc.addupdate_scatter(lsum_v, [k], sv, mask=live)
            plsc.addupdate_scatter(lcnt_v, [k], cv, mask=live)

    d1 = pltpu.async_copy(lsum_v, sums_hbm.at[cid, sid, :], out_sem)
    d2 = pltpu.async_copy(lcnt_v, cnts_hbm.at[cid, sid, :], out_sem)
    d1.wait()
    d2.wait()


_sc_call = pl.kernel(
    _sc_body,
    out_type=(jax.ShapeDtypeStruct((NC, NS, NB), jnp.float32),
              jax.ShapeDtypeStruct((NC, NS, NB), jnp.float32)),
    mesh=plsc.VectorSubcoreMesh(core_axis_name="c", subcore_axis_name="s"),
    compiler_params=pltpu.CompilerParams(needs_layout_passes=False),
    scratch_types=[
        pltpu.VMEM((SUB,), jnp.float32),
        pltpu.VMEM((SUB,), jnp.int32),
        pltpu.VMEM((SUB,), jnp.int32),
        pltpu.VMEM((SUB,), jnp.float32),
        pltpu.VMEM((SUB,), jnp.int32),
        pltpu.VMEM((SUB,), jnp.int32),
        pltpu.VMEM((NB,), jnp.float32),
        pltpu.VMEM((NB,), jnp.float32),
        pltpu.SemaphoreType.DMA,
        pltpu.SemaphoreType.DMA,
        pltpu.SemaphoreType.DMA,
    ],
)


def _bce_body(scores_ref, labels_ref, out_ref):
    i = pl.program_id(0)

    @pl.when(i == 0)
    def _():
        out_ref[0, 0] = 0.0

    s = scores_ref[...]
    lb = labels_ref[...].astype(jnp.float32)
    t = lb * jnp.maximum(jnp.log(s), -100.0) \
        + (1.0 - lb) * jnp.maximum(jnp.log1p(-s), -100.0)
    out_ref[0, 0] += jnp.sum(t)


def _bce_call(scores2, labels2):
    return pl.pallas_call(
        _bce_body,
        grid=(GRID,),
        in_specs=[
            pl.BlockSpec((BROWS, COLS), lambda i: (i, 0)),
            pl.BlockSpec((BROWS, COLS), lambda i: (i, 0)),
        ],
        out_specs=pl.BlockSpec(memory_space=pltpu.SMEM),
        out_shape=jax.ShapeDtypeStruct((1, 1), jnp.float32),
    )(scores2, labels2)


def _comb_body(sums_ref, cnts_ref, bce_ref, out_ref):
    sums = jnp.sum(sums_ref[...], axis=0, keepdims=True)
    cnts = jnp.sum(cnts_ref[...], axis=0, keepdims=True)
    neg_s = sums[:, :NQ]
    pos_s = sums[:, NQ:]
    neg_c = cnts[:, :NQ]
    pos_c = cnts[:, NQ:]
    valid = (pos_c > 0.0) & (neg_c > 0.0)
    pos_m = pos_s / jnp.maximum(pos_c, 1.0)
    neg_m = neg_s / jnp.maximum(neg_c, 1.0)
    delta = pos_m - neg_m
    terms = jnp.where(valid, jnp.maximum(GAMMA - delta, 0.0), 0.0)
    ng = jnp.sum(valid.astype(jnp.float32))
    ldc = jnp.where(ng > 0.0, jnp.sum(terms) / jnp.maximum(ng, 1.0), 0.0)
    lce = -bce_ref[0, 0] / N
    out_ref[0, 0] = lce + LAM * ldc


def _comb_call(sums_p, cnts_p, bce):
    return pl.pallas_call(
        _comb_body,
        in_specs=[
            pl.BlockSpec((NC * NS, NB), lambda: (0, 0)),
            pl.BlockSpec((NC * NS, NB), lambda: (0, 0)),
            pl.BlockSpec(memory_space=pltpu.SMEM),
        ],
        out_specs=pl.BlockSpec(memory_space=pltpu.SMEM),
        out_shape=jax.ShapeDtypeStruct((1, 1), jnp.float32),
    )(sums_p, cnts_p, bce)


def kernel(scores, labels, qids):
    labels_i = labels.astype(jnp.int32)
    qids_i = qids.astype(jnp.int32)
    sums_o, cnts_o = _sc_call(scores, labels_i, qids_i)
    sums_p = sums_o.reshape(NC * NS, NB)
    cnts_p = cnts_o.reshape(NC * NS, NB)
    scores2 = scores.reshape(ROWS, COLS)
    labels2 = labels_i.reshape(ROWS, COLS)
    bce = _bce_call(scores2, labels2)
    out = _comb_call(sums_p, cnts_p, bce)
    return out[0, 0]
